# baseline (speedup 1.0000x reference)
import functools

import jax
import jax.numpy as jnp
from jax import lax
from jax.experimental import pallas as pl
from jax.experimental.pallas import tpu as pltpu
from jax.experimental.pallas import tpu_sc as plsc

N = 10000
E = 320000
ETOT = E + N
D_IN = 128
H1 = 256
H2 = 128
NCLS = 7
NEG = 0.2

NCORE = 2
NSUB = 16
NW = NCORE * NSUB
LANES = 16

CH = 64
NCHUNK = 164
EPT = CH * NCHUNK
EPAD = EPT * NW
BLK = 1000
NB = N // BLK

_MM = functools.partial(jnp.dot, preferred_element_type=jnp.float32)


def _elu(v):
    return jnp.where(v > 0, v, jnp.exp(jnp.minimum(v, 0.0)) - 1.0)


def _pre1_body(x_ref, W1_ref, a1s_ref, a1d_ref, as_ref, ad_ref, c_ref, m_ref):
    i = pl.program_id(0)
    us = _MM(W1_ref[...], a1s_ref[...])
    ud = _MM(W1_ref[...], a1d_ref[...])
    xb = x_ref[...]
    asb = _MM(xb, us)
    adb = _MM(xb, ud)
    as_ref[...] = asb
    ad_ref[...] = adb
    bs, bd = jnp.max(asb), jnp.max(adb)

    @pl.when(i == 0)
    def _():
        m_ref[0] = bs
        m_ref[1] = bd

    @pl.when(i > 0)
    def _():
        m_ref[0] = jnp.maximum(m_ref[0], bs)
        m_ref[1] = jnp.maximum(m_ref[1], bd)

    @pl.when(i == NB - 1)
    def _():
        s = m_ref[0] + m_ref[1]
        c = jnp.where(s >= 0, s, NEG * s)
        c_ref[...] = jnp.full((1, 128), c, jnp.float32)


def _pre1(x, W1, a1s, a1d):
    return pl.pallas_call(
        _pre1_body,
        grid=(NB,),
        in_specs=[
            pl.BlockSpec((BLK, D_IN), lambda i: (i, 0)),
            pl.BlockSpec((D_IN, H1), lambda i: (0, 0)),
            pl.BlockSpec((H1, 1), lambda i: (0, 0)),
            pl.BlockSpec((H1, 1), lambda i: (0, 0)),
        ],
        out_specs=[
            pl.BlockSpec((BLK, 1), lambda i: (i, 0)),
            pl.BlockSpec((BLK, 1), lambda i: (i, 0)),
            pl.BlockSpec((1, 128), lambda i: (0, 0)),
        ],
        out_shape=[
            jax.ShapeDtypeStruct((N, 1), jnp.float32),
            jax.ShapeDtypeStruct((N, 1), jnp.float32),
            jax.ShapeDtypeStruct((1, 128), jnp.float32),
        ],
        scratch_shapes=[pltpu.SMEM((2,), jnp.float32)],
    )(x, W1, a1s[:, None], a1d[:, None])


def _mid1_body(agg_ref, den_ref, W1_ref, b1_ref, W2_ref, a2s_ref, a2d_ref,
               g2_ref, as_ref, ad_ref, c_ref, m_ref):
    i = pl.program_id(0)
    agg = agg_ref[0] + agg_ref[1]
    den = jnp.sum(den_ref[...], axis=0) + 1e-16
    h = _elu(_MM(agg / den, W1_ref[...]) + b1_ref[...])
    g2 = _MM(h, W2_ref[...])
    g2_ref[...] = g2
    asb = _MM(g2, a2s_ref[...])
    adb = _MM(g2, a2d_ref[...])
    as_ref[...] = asb
    ad_ref[...] = adb
    bs, bd = jnp.max(asb), jnp.max(adb)

    @pl.when(i == 0)
    def _():
        m_ref[0] = bs
        m_ref[1] = bd

    @pl.when(i > 0)
    def _():
        m_ref[0] = jnp.maximum(m_ref[0], bs)
        m_ref[1] = jnp.maximum(m_ref[1], bd)

    @pl.when(i == NB - 1)
    def _():
        s = m_ref[0] + m_ref[1]
        c = jnp.where(s >= 0, s, NEG * s)
        c_ref[...] = jnp.full((1, 128), c, jnp.float32)


def _mid1(agg, den, W1, b1, W2, a2s, a2d):
    return pl.pallas_call(
        _mid1_body,
        grid=(NB,),
        in_specs=[
            pl.BlockSpec((2, BLK, D_IN), lambda i: (0, i, 0)),
            pl.BlockSpec((NW, BLK, 1), lambda i: (0, i, 0)),
            pl.BlockSpec((D_IN, H1), lambda i: (0, 0)),
            pl.BlockSpec((1, H1), lambda i: (0, 0)),
            pl.BlockSpec((H1, H2), lambda i: (0, 0)),
            pl.BlockSpec((H2, 1), lambda i: (0, 0)),
            pl.BlockSpec((H2, 1), lambda i: (0, 0)),
        ],
        out_specs=[
            pl.BlockSpec((BLK, H2), lambda i: (i, 0)),
            pl.BlockSpec((BLK, 1), lambda i: (i, 0)),
            pl.BlockSpec((BLK, 1), lambda i: (i, 0)),
            pl.BlockSpec((1, 128), lambda i: (0, 0)),
        ],
        out_shape=[
            jax.ShapeDtypeStruct((N, H2), jnp.float32),
            jax.ShapeDtypeStruct((N, 1), jnp.float32),
            jax.ShapeDtypeStruct((N, 1), jnp.float32),
            jax.ShapeDtypeStruct((1, 128), jnp.float32),
        ],
        scratch_shapes=[pltpu.SMEM((2,), jnp.float32)],
    )(agg, den[..., None], W1, b1[None, :], W2, a2s[:, None], a2d[:, None])


def _mid2_body(agg_ref, den_ref, b2_ref, W3_ref, a3s_ref, a3d_ref,
               g3_ref, as_ref, ad_ref, c_ref, m_ref):
    i = pl.program_id(0)
    agg = agg_ref[0] + agg_ref[1]
    den = jnp.sum(den_ref[...], axis=0) + 1e-16
    h = _elu(agg / den + b2_ref[...])
    g3 = _MM(h, W3_ref[...])
    g3_ref[...] = g3
    asb = _MM(g3, a3s_ref[...])
    adb = _MM(g3, a3d_ref[...])
    as_ref[...] = asb
    ad_ref[...] = adb
    bs, bd = jnp.max(asb), jnp.max(adb)

    @pl.when(i == 0)
    def _():
        m_ref[0] = bs
        m_ref[1] = bd

    @pl.when(i > 0)
    def _():
        m_ref[0] = jnp.maximum(m_ref[0], bs)
        m_ref[1] = jnp.maximum(m_ref[1], bd)

    @pl.when(i == NB - 1)
    def _():
        s = m_ref[0] + m_ref[1]
        c = jnp.where(s >= 0, s, NEG * s)
        c_ref[...] = jnp.full((1, 128), c, jnp.float32)


def _mid2(agg, den, b2, W3p, a3sp, a3dp):
    return pl.pallas_call(
        _mid2_body,
        grid=(NB,),
        in_specs=[
            pl.BlockSpec((2, BLK, H2), lambda i: (0, i, 0)),
            pl.BlockSpec((NW, BLK, 1), lambda i: (0, i, 0)),
            pl.BlockSpec((1, H2), lambda i: (0, 0)),
            pl.BlockSpec((H2, 16), lambda i: (0, 0)),
            pl.BlockSpec((16, 1), lambda i: (0, 0)),
            pl.BlockSpec((16, 1), lambda i: (0, 0)),
        ],
        out_specs=[
            pl.BlockSpec((BLK, 16), lambda i: (i, 0)),
            pl.BlockSpec((BLK, 1), lambda i: (i, 0)),
            pl.BlockSpec((BLK, 1), lambda i: (i, 0)),
            pl.BlockSpec((1, 128), lambda i: (0, 0)),
        ],
        out_shape=[
            jax.ShapeDtypeStruct((N, 16), jnp.float32),
            jax.ShapeDtypeStruct((N, 1), jnp.float32),
            jax.ShapeDtypeStruct((N, 1), jnp.float32),
            jax.ShapeDtypeStruct((1, 128), jnp.float32),
        ],
        scratch_shapes=[pltpu.SMEM((2,), jnp.float32)],
    )(agg, den[..., None], b2[None, :], W3p, a3sp[:, None], a3dp[:, None])


def _fin_body(agg_ref, den_ref, b3_ref, out_ref):
    agg = agg_ref[0] + agg_ref[1]
    den = jnp.sum(den_ref[...], axis=0) + 1e-16
    out_ref[...] = agg / den + b3_ref[...]


def _fin(agg, den, b3p):
    return pl.pallas_call(
        _fin_body,
        grid=(NB,),
        in_specs=[
            pl.BlockSpec((2, BLK, 16), lambda i: (0, i, 0)),
            pl.BlockSpec((NW, BLK, 1), lambda i: (0, i, 0)),
            pl.BlockSpec((1, 16), lambda i: (0, 0)),
        ],
        out_specs=pl.BlockSpec((BLK, 16), lambda i: (i, 0)),
        out_shape=jax.ShapeDtypeStruct((N, 16), jnp.float32),
    )(agg, den[..., None], b3p[None, :])


def _sc_edge_layer(F):
    mesh = plsc.VectorSubcoreMesh(core_axis_name="c", subcore_axis_name="s")
    rpt = N // NSUB

    @functools.partial(
        pl.kernel,
        out_type=[
            jax.ShapeDtypeStruct((NCORE, N, F), jnp.float32),
            jax.ShapeDtypeStruct((NW, N), jnp.float32),
        ],
        mesh=mesh,
        compiler_params=pltpu.CompilerParams(use_tc_tiling_on_sc=False,
                                             needs_layout_passes=False),
        scratch_types=[
            pltpu.VMEM_SHARED((N, F), jnp.float32),
            pltpu.VMEM((N + 16,), jnp.float32),
            pltpu.VMEM((N + 16,), jnp.float32),
            pltpu.VMEM((N,), jnp.float32),
            pltpu.VMEM((16,), jnp.float32),
            pltpu.VMEM((CH,), jnp.int32),
            pltpu.VMEM((CH,), jnp.int32),
            pltpu.VMEM((CH,), jnp.int32),
            pltpu.VMEM((CH,), jnp.int32),
            pltpu.VMEM((CH,), jnp.float32),
            pltpu.VMEM((CH, F), jnp.float32),
            pltpu.VMEM((CH, F), jnp.float32),
            pltpu.SemaphoreType.DMA,
            pltpu.SemaphoreType.DMA,
            pltpu.SemaphoreType.DMA,
            pltpu.SemaphoreType.DMA,
        ],
    )
    def sc_fn(src_hbm, dst_hbm, as_hbm, ad_hbm, c_hbm, g_hbm, zrows_hbm,
              zden_hbm, agg_out, den_out,
              agg_sh, as_v, ad_v, den_v, c_v, src_a, src_b, dst_a, dst_b,
              w_v, rows_a, rows_b,
              sem_ra, sem_rb, sem_sa, sem_sb):
        cid = lax.axis_index("c")
        sid = lax.axis_index("s")
        wid = sid * NCORE + cid
        base = wid * EPT

        pltpu.sync_copy(as_hbm, as_v)
        pltpu.sync_copy(ad_hbm, ad_v)
        pltpu.sync_copy(c_hbm, c_v)
        pltpu.sync_copy(zden_hbm, den_v)
        pltpu.sync_copy(zrows_hbm.at[pl.ds(sid * rpt, rpt)],
                        agg_sh.at[pl.ds(sid * rpt, rpt)])
        plsc.subcore_barrier()

        cvec = c_v[...]

        def prefetch(k, sv, dv, rv, sem_r):
            off = base + k * CH
            pltpu.sync_copy(src_hbm.at[pl.ds(off, CH)], sv)
            pltpu.sync_copy(dst_hbm.at[pl.ds(off, CH)], dv)
            pltpu.async_copy(g_hbm.at[sv], rv, sem_r)

        def weight_pass(sv, dv):
            for g16 in range(CH // LANES):
                sl = pl.ds(g16 * LANES, LANES)
                didx = dv[sl]
                s = (plsc.load_gather(as_v, [sv[sl]])
                     + plsc.load_gather(ad_v, [didx]))
                e = jnp.where(s >= 0, s, NEG * s)
                w = jnp.exp(e - cvec)
                w_v[sl] = w
                plsc.addupdate_scatter(den_v, [didx], w)

        def scale_scatter(sv, dv, rv, sem_r, sem_s):
            pltpu.make_async_copy(g_hbm.at[sv], rv, sem_r).wait()
            for row in range(CH):
                ws = plsc.load_gather(
                    w_v, [jnp.full((LANES,), row, jnp.int32)])
                for f in range(F // LANES):
                    fsl = pl.ds(f * LANES, LANES)
                    rv[row, fsl] = rv[row, fsl] * ws
            pltpu.async_copy(rv, agg_sh.at[dv], sem_s, add=True)

        def wait_scatter(dv, rv, sem_s):
            pltpu.make_async_copy(rv, agg_sh.at[dv], sem_s).wait()

        prefetch(0, src_a, dst_a, rows_a, sem_ra)

        def pair(j, carry):
            k0 = 2 * j

            @pl.when(j > 0)
            def _():
                wait_scatter(dst_b, rows_b, sem_sb)

            prefetch(k0 + 1, src_b, dst_b, rows_b, sem_rb)
            weight_pass(src_a, dst_a)
            scale_scatter(src_a, dst_a, rows_a, sem_ra, sem_sa)

            weight_pass(src_b, dst_b)

            @pl.when(k0 + 2 < NCHUNK)
            def _():
                wait_scatter(dst_a, rows_a, sem_sa)
                prefetch(k0 + 2, src_a, dst_a, rows_a, sem_ra)

            scale_scatter(src_b, dst_b, rows_b, sem_rb, sem_sb)
            return carry

        lax.fori_loop(0, NCHUNK // 2, pair, 0)
        wait_scatter(dst_a, rows_a, sem_sa)
        wait_scatter(dst_b, rows_b, sem_sb)

        pltpu.sync_copy(den_v, den_out.at[wid])
        plsc.subcore_barrier()

        rsl = pl.ds(sid * rpt, rpt)
        pltpu.sync_copy(agg_sh.at[rsl], agg_out.at[cid, rsl])

    return sc_fn


_sc128 = _sc_edge_layer(128)
_sc16 = _sc_edge_layer(16)


_SENT = -1e30


def _pad_tables(asv, adv):
    a = jnp.concatenate([asv, jnp.full((16,), _SENT, jnp.float32)])
    b = jnp.concatenate([adv, jnp.zeros((16,), jnp.float32)])
    return a, b


def _pad_g(g, f):
    return jnp.concatenate([g, jnp.zeros((16, f), jnp.float32)])


def kernel(x, edge_index, W1, a1s, a1d, b1, W2, a2s, a2d, b2,
           W3, a3s, a3d, b3):
    loop = jnp.arange(N, dtype=edge_index.dtype)
    pad_src = jnp.full((EPAD - ETOT,), N, edge_index.dtype)
    pad_dst = jnp.zeros((EPAD - ETOT,), edge_index.dtype)
    src = jnp.concatenate([edge_index[0], loop, pad_src])
    dst = jnp.concatenate([edge_index[1], loop, pad_dst])
    order = jnp.argsort(dst)
    totc = EPAD // CH
    src = src[order].reshape(CH, totc).T.reshape(EPAD)
    dst = dst[order].reshape(CH, totc).T.reshape(EPAD)
    z128 = jnp.zeros((N, 128), jnp.float32)
    z16 = jnp.zeros((N, 16), jnp.float32)
    zden = jnp.zeros((N,), jnp.float32)

    as1, ad1, c1 = _pre1(x, W1, a1s, a1d)
    asp, adp = _pad_tables(as1.reshape(N), ad1.reshape(N))
    agg1, den1 = _sc128(src, dst, asp, adp,
                        c1.reshape(128)[:16], _pad_g(x, 128), z128, zden)

    g2, as2, ad2, c2 = _mid1(agg1, den1, W1, b1, W2, a2s, a2d)
    asp, adp = _pad_tables(as2.reshape(N), ad2.reshape(N))
    agg2, den2 = _sc128(src, dst, asp, adp,
                        c2.reshape(128)[:16], _pad_g(g2, 128), z128, zden)

    W3p = jnp.pad(W3, ((0, 0), (0, 16 - NCLS)))
    a3sp = jnp.pad(a3s, (0, 16 - NCLS))
    a3dp = jnp.pad(a3d, (0, 16 - NCLS))
    b3p = jnp.pad(b3, (0, 16 - NCLS))
    g3, as3, ad3, c3 = _mid2(agg2, den2, b2, W3p, a3sp, a3dp)
    asp, adp = _pad_tables(as3.reshape(N), ad3.reshape(N))
    agg3, den3 = _sc16(src, dst, asp, adp,
                       c3.reshape(128)[:16], _pad_g(g3, 16), z16, zden)

    out = _fin(agg3, den3, b3p)
    return out[:, :NCLS]


# speedup vs baseline: 1.3583x; 1.3583x over previous
import functools

import jax
import jax.numpy as jnp
from jax import lax
from jax.experimental import pallas as pl
from jax.experimental.pallas import tpu as pltpu
from jax.experimental.pallas import tpu_sc as plsc

N = 10000
E = 320000
ETOT = E + N
D_IN = 128
H1 = 256
H2 = 128
NCLS = 7
NEG = 0.2

NCORE = 2
NSUB = 16
NW = NCORE * NSUB
LANES = 16

CH = 64
NCHUNK = 164
EPT = CH * NCHUNK
EPAD = EPT * NW
BLK = 1000
NB = N // BLK

_MM = functools.partial(jnp.dot, preferred_element_type=jnp.float32)


def _elu(v):
    return jnp.where(v > 0, v, jnp.exp(jnp.minimum(v, 0.0)) - 1.0)


def _pre1_body(x_ref, W1_ref, a1s_ref, a1d_ref, as_ref, ad_ref, c_ref, m_ref):
    i = pl.program_id(0)
    us = _MM(W1_ref[...], a1s_ref[...])
    ud = _MM(W1_ref[...], a1d_ref[...])
    xb = x_ref[...]
    asb = _MM(xb, us)
    adb = _MM(xb, ud)
    as_ref[...] = asb
    ad_ref[...] = adb
    bs, bd = jnp.max(asb), jnp.max(adb)

    @pl.when(i == 0)
    def _():
        m_ref[0] = bs
        m_ref[1] = bd

    @pl.when(i > 0)
    def _():
        m_ref[0] = jnp.maximum(m_ref[0], bs)
        m_ref[1] = jnp.maximum(m_ref[1], bd)

    @pl.when(i == NB - 1)
    def _():
        s = m_ref[0] + m_ref[1]
        c = jnp.where(s >= 0, s, NEG * s)
        c_ref[...] = jnp.full((1, 128), c, jnp.float32)


def _pre1(x, W1, a1s, a1d):
    return pl.pallas_call(
        _pre1_body,
        grid=(NB,),
        in_specs=[
            pl.BlockSpec((BLK, D_IN), lambda i: (i, 0)),
            pl.BlockSpec((D_IN, H1), lambda i: (0, 0)),
            pl.BlockSpec((H1, 1), lambda i: (0, 0)),
            pl.BlockSpec((H1, 1), lambda i: (0, 0)),
        ],
        out_specs=[
            pl.BlockSpec((BLK, 1), lambda i: (i, 0)),
            pl.BlockSpec((BLK, 1), lambda i: (i, 0)),
            pl.BlockSpec((1, 128), lambda i: (0, 0)),
        ],
        out_shape=[
            jax.ShapeDtypeStruct((N, 1), jnp.float32),
            jax.ShapeDtypeStruct((N, 1), jnp.float32),
            jax.ShapeDtypeStruct((1, 128), jnp.float32),
        ],
        scratch_shapes=[pltpu.SMEM((2,), jnp.float32)],
    )(x, W1, a1s[:, None], a1d[:, None])


def _mid1_body(agg_ref, den_ref, W1_ref, b1_ref, W2_ref, a2s_ref, a2d_ref,
               g2_ref, as_ref, ad_ref, c_ref, m_ref):
    i = pl.program_id(0)
    agg = agg_ref[0] + agg_ref[1]
    den = jnp.sum(den_ref[0], axis=0).reshape(BLK, 1) + 1e-16
    h = _elu(_MM(agg / den, W1_ref[...]) + b1_ref[...])
    g2 = _MM(h, W2_ref[...])
    g2_ref[...] = g2
    asb = _MM(g2, a2s_ref[...])
    adb = _MM(g2, a2d_ref[...])
    as_ref[...] = asb
    ad_ref[...] = adb
    bs, bd = jnp.max(asb), jnp.max(adb)

    @pl.when(i == 0)
    def _():
        m_ref[0] = bs
        m_ref[1] = bd

    @pl.when(i > 0)
    def _():
        m_ref[0] = jnp.maximum(m_ref[0], bs)
        m_ref[1] = jnp.maximum(m_ref[1], bd)

    @pl.when(i == NB - 1)
    def _():
        s = m_ref[0] + m_ref[1]
        c = jnp.where(s >= 0, s, NEG * s)
        c_ref[...] = jnp.full((1, 128), c, jnp.float32)


def _mid1(agg, den, W1, b1, W2, a2s, a2d):
    return pl.pallas_call(
        _mid1_body,
        grid=(NB,),
        in_specs=[
            pl.BlockSpec((2, BLK, D_IN), lambda i: (0, i, 0)),
            pl.BlockSpec((1, NW, BLK), lambda i: (i, 0, 0)),
            pl.BlockSpec((D_IN, H1), lambda i: (0, 0)),
            pl.BlockSpec((1, H1), lambda i: (0, 0)),
            pl.BlockSpec((H1, H2), lambda i: (0, 0)),
            pl.BlockSpec((H2, 1), lambda i: (0, 0)),
            pl.BlockSpec((H2, 1), lambda i: (0, 0)),
        ],
        out_specs=[
            pl.BlockSpec((BLK, H2), lambda i: (i, 0)),
            pl.BlockSpec((BLK, 1), lambda i: (i, 0)),
            pl.BlockSpec((BLK, 1), lambda i: (i, 0)),
            pl.BlockSpec((1, 128), lambda i: (0, 0)),
        ],
        out_shape=[
            jax.ShapeDtypeStruct((N, H2), jnp.float32),
            jax.ShapeDtypeStruct((N, 1), jnp.float32),
            jax.ShapeDtypeStruct((N, 1), jnp.float32),
            jax.ShapeDtypeStruct((1, 128), jnp.float32),
        ],
        scratch_shapes=[pltpu.SMEM((2,), jnp.float32)],
    )(agg, jnp.swapaxes(den.reshape(NW, NB, BLK), 0, 1), W1, b1[None, :], W2,
      a2s[:, None], a2d[:, None])


def _mid2_body(agg_ref, den_ref, b2_ref, W3_ref, a3s_ref, a3d_ref,
               g3_ref, as_ref, ad_ref, c_ref, m_ref):
    i = pl.program_id(0)
    agg = agg_ref[0] + agg_ref[1]
    den = jnp.sum(den_ref[0], axis=0).reshape(BLK, 1) + 1e-16
    h = _elu(agg / den + b2_ref[...])
    g3 = _MM(h, W3_ref[...])
    g3_ref[...] = g3
    asb = _MM(g3, a3s_ref[...])
    adb = _MM(g3, a3d_ref[...])
    as_ref[...] = asb
    ad_ref[...] = adb
    bs, bd = jnp.max(asb), jnp.max(adb)

    @pl.when(i == 0)
    def _():
        m_ref[0] = bs
        m_ref[1] = bd

    @pl.when(i > 0)
    def _():
        m_ref[0] = jnp.maximum(m_ref[0], bs)
        m_ref[1] = jnp.maximum(m_ref[1], bd)

    @pl.when(i == NB - 1)
    def _():
        s = m_ref[0] + m_ref[1]
        c = jnp.where(s >= 0, s, NEG * s)
        c_ref[...] = jnp.full((1, 128), c, jnp.float32)


def _mid2(agg, den, b2, W3p, a3sp, a3dp):
    return pl.pallas_call(
        _mid2_body,
        grid=(NB,),
        in_specs=[
            pl.BlockSpec((2, BLK, H2), lambda i: (0, i, 0)),
            pl.BlockSpec((1, NW, BLK), lambda i: (i, 0, 0)),
            pl.BlockSpec((1, H2), lambda i: (0, 0)),
            pl.BlockSpec((H2, 16), lambda i: (0, 0)),
            pl.BlockSpec((16, 1), lambda i: (0, 0)),
            pl.BlockSpec((16, 1), lambda i: (0, 0)),
        ],
        out_specs=[
            pl.BlockSpec((BLK, 16), lambda i: (i, 0)),
            pl.BlockSpec((BLK, 1), lambda i: (i, 0)),
            pl.BlockSpec((BLK, 1), lambda i: (i, 0)),
            pl.BlockSpec((1, 128), lambda i: (0, 0)),
        ],
        out_shape=[
            jax.ShapeDtypeStruct((N, 16), jnp.float32),
            jax.ShapeDtypeStruct((N, 1), jnp.float32),
            jax.ShapeDtypeStruct((N, 1), jnp.float32),
            jax.ShapeDtypeStruct((1, 128), jnp.float32),
        ],
        scratch_shapes=[pltpu.SMEM((2,), jnp.float32)],
    )(agg, jnp.swapaxes(den.reshape(NW, NB, BLK), 0, 1), b2[None, :], W3p,
      a3sp[:, None], a3dp[:, None])


def _fin_body(agg_ref, den_ref, b3_ref, out_ref):
    agg = agg_ref[0] + agg_ref[1]
    den = jnp.sum(den_ref[0], axis=0).reshape(BLK, 1) + 1e-16
    out_ref[...] = agg / den + b3_ref[...]


def _fin(agg, den, b3p):
    return pl.pallas_call(
        _fin_body,
        grid=(NB,),
        in_specs=[
            pl.BlockSpec((2, BLK, 16), lambda i: (0, i, 0)),
            pl.BlockSpec((1, NW, BLK), lambda i: (i, 0, 0)),
            pl.BlockSpec((1, 16), lambda i: (0, 0)),
        ],
        out_specs=pl.BlockSpec((BLK, 16), lambda i: (i, 0)),
        out_shape=jax.ShapeDtypeStruct((N, 16), jnp.float32),
    )(agg, jnp.swapaxes(den.reshape(NW, NB, BLK), 0, 1), b3p[None, :])


def _sc_edge_layer(F):
    mesh = plsc.VectorSubcoreMesh(core_axis_name="c", subcore_axis_name="s")
    rpt = N // NSUB

    @functools.partial(
        pl.kernel,
        out_type=[
            jax.ShapeDtypeStruct((NCORE, N, F), jnp.float32),
            jax.ShapeDtypeStruct((NW, N), jnp.float32),
        ],
        mesh=mesh,
        compiler_params=pltpu.CompilerParams(use_tc_tiling_on_sc=False,
                                             needs_layout_passes=False),
        scratch_types=[
            pltpu.VMEM_SHARED((N, F), jnp.float32),
            pltpu.VMEM((N + 16,), jnp.float32),
            pltpu.VMEM((N + 16,), jnp.float32),
            pltpu.VMEM((N,), jnp.float32),
            pltpu.VMEM((16,), jnp.float32),
            pltpu.VMEM((CH,), jnp.int32),
            pltpu.VMEM((CH,), jnp.int32),
            pltpu.VMEM((CH,), jnp.int32),
            pltpu.VMEM((CH,), jnp.int32),
            pltpu.VMEM((CH,), jnp.float32),
            pltpu.VMEM((CH, F), jnp.float32),
            pltpu.VMEM((CH, F), jnp.float32),
            pltpu.SemaphoreType.DMA,
            pltpu.SemaphoreType.DMA,
            pltpu.SemaphoreType.DMA,
            pltpu.SemaphoreType.DMA,
        ],
    )
    def sc_fn(src_hbm, dst_hbm, as_hbm, ad_hbm, c_hbm, g_hbm, zrows_hbm,
              zden_hbm, agg_out, den_out,
              agg_sh, as_v, ad_v, den_v, c_v, src_a, src_b, dst_a, dst_b,
              w_v, rows_a, rows_b,
              sem_ra, sem_rb, sem_sa, sem_sb):
        cid = lax.axis_index("c")
        sid = lax.axis_index("s")
        wid = sid * NCORE + cid
        base = wid * EPT

        pltpu.sync_copy(as_hbm, as_v)
        pltpu.sync_copy(ad_hbm, ad_v)
        pltpu.sync_copy(c_hbm, c_v)
        pltpu.sync_copy(zden_hbm, den_v)
        pltpu.sync_copy(zrows_hbm.at[pl.ds(sid * rpt, rpt)],
                        agg_sh.at[pl.ds(sid * rpt, rpt)])
        plsc.subcore_barrier()

        cvec = c_v[...]

        def prefetch(k, sv, dv, rv, sem_r):
            off = base + k * CH
            pltpu.sync_copy(src_hbm.at[pl.ds(off, CH)], sv)
            pltpu.sync_copy(dst_hbm.at[pl.ds(off, CH)], dv)
            pltpu.async_copy(g_hbm.at[sv], rv, sem_r)

        def weight_pass(sv, dv):
            for g16 in range(CH // LANES):
                sl = pl.ds(g16 * LANES, LANES)
                didx = dv[sl]
                s = (plsc.load_gather(as_v, [sv[sl]])
                     + plsc.load_gather(ad_v, [didx]))
                e = jnp.where(s >= 0, s, NEG * s)
                w = jnp.exp(e - cvec)
                w_v[sl] = w
                plsc.addupdate_scatter(den_v, [didx], w)

        def scale_scatter(sv, dv, rv, sem_r, sem_s):
            pltpu.make_async_copy(g_hbm.at[sv], rv, sem_r).wait()
            for row in range(CH):
                ws = plsc.load_gather(
                    w_v, [jnp.full((LANES,), row, jnp.int32)])
                for f in range(F // LANES):
                    fsl = pl.ds(f * LANES, LANES)
                    rv[row, fsl] = rv[row, fsl] * ws
            pltpu.async_copy(rv, agg_sh.at[dv], sem_s, add=True)

        def wait_scatter(dv, rv, sem_s):
            pltpu.make_async_copy(rv, agg_sh.at[dv], sem_s).wait()

        prefetch(0, src_a, dst_a, rows_a, sem_ra)

        def pair(j, carry):
            k0 = 2 * j

            @pl.when(j > 0)
            def _():
                wait_scatter(dst_b, rows_b, sem_sb)

            prefetch(k0 + 1, src_b, dst_b, rows_b, sem_rb)
            weight_pass(src_a, dst_a)
            scale_scatter(src_a, dst_a, rows_a, sem_ra, sem_sa)

            weight_pass(src_b, dst_b)

            @pl.when(k0 + 2 < NCHUNK)
            def _():
                wait_scatter(dst_a, rows_a, sem_sa)
                prefetch(k0 + 2, src_a, dst_a, rows_a, sem_ra)

            scale_scatter(src_b, dst_b, rows_b, sem_rb, sem_sb)
            return carry

        lax.fori_loop(0, NCHUNK // 2, pair, 0)
        wait_scatter(dst_a, rows_a, sem_sa)
        wait_scatter(dst_b, rows_b, sem_sb)

        pltpu.sync_copy(den_v, den_out.at[wid])
        plsc.subcore_barrier()

        rsl = pl.ds(sid * rpt, rpt)
        pltpu.sync_copy(agg_sh.at[rsl], agg_out.at[cid, rsl])

    return sc_fn


_sc128 = _sc_edge_layer(128)
_sc16 = _sc_edge_layer(16)


_SENT = -1e30


def _pad_tables(asv, adv):
    a = jnp.concatenate([asv, jnp.full((16,), _SENT, jnp.float32)])
    b = jnp.concatenate([adv, jnp.zeros((16,), jnp.float32)])
    return a, b


def _pad_g(g, f):
    return jnp.concatenate([g, jnp.zeros((16, f), jnp.float32)])


def kernel(x, edge_index, W1, a1s, a1d, b1, W2, a2s, a2d, b2,
           W3, a3s, a3d, b3):
    loop = jnp.arange(N, dtype=edge_index.dtype)
    pad_src = jnp.full((EPAD - ETOT,), N, edge_index.dtype)
    pad_dst = jnp.zeros((EPAD - ETOT,), edge_index.dtype)
    src = jnp.concatenate([edge_index[0], loop, pad_src])
    dst = jnp.concatenate([edge_index[1], loop, pad_dst])
    order = jnp.argsort(dst)
    totc = EPAD // CH
    src = src[order].reshape(CH, totc).T.reshape(EPAD)
    dst = dst[order].reshape(CH, totc).T.reshape(EPAD)
    z128 = jnp.zeros((N, 128), jnp.float32)
    z16 = jnp.zeros((N, 16), jnp.float32)
    zden = jnp.zeros((N,), jnp.float32)

    as1, ad1, c1 = _pre1(x, W1, a1s, a1d)
    asp, adp = _pad_tables(as1.reshape(N), ad1.reshape(N))
    agg1, den1 = _sc128(src, dst, asp, adp,
                        c1.reshape(128)[:16], _pad_g(x, 128), z128, zden)

    g2, as2, ad2, c2 = _mid1(agg1, den1, W1, b1, W2, a2s, a2d)
    asp, adp = _pad_tables(as2.reshape(N), ad2.reshape(N))
    agg2, den2 = _sc128(src, dst, asp, adp,
                        c2.reshape(128)[:16], _pad_g(g2, 128), z128, zden)

    W3p = jnp.pad(W3, ((0, 0), (0, 16 - NCLS)))
    a3sp = jnp.pad(a3s, (0, 16 - NCLS))
    a3dp = jnp.pad(a3d, (0, 16 - NCLS))
    b3p = jnp.pad(b3, (0, 16 - NCLS))
    g3, as3, ad3, c3 = _mid2(agg2, den2, b2, W3p, a3sp, a3dp)
    asp, adp = _pad_tables(as3.reshape(N), ad3.reshape(N))
    agg3, den3 = _sc16(src, dst, asp, adp,
                       c3.reshape(128)[:16], _pad_g(g3, 16), z16, zden)

    out = _fin(agg3, den3, b3p)
    return out[:, :NCLS]


# speedup vs baseline: 1.3964x; 1.0281x over previous
import functools

import jax
import jax.numpy as jnp
from jax import lax
from jax.experimental import pallas as pl
from jax.experimental.pallas import tpu as pltpu
from jax.experimental.pallas import tpu_sc as plsc

N = 10000
E = 320000
ETOT = E + N
D_IN = 128
H1 = 256
H2 = 128
NCLS = 7
NEG = 0.2

NCORE = 2
NSUB = 16
NW = NCORE * NSUB
LANES = 16

CH = 64
NCHUNK = 164
EPT = CH * NCHUNK
EPAD = EPT * NW
BLK = 1000
NB = N // BLK

_MM = functools.partial(jnp.dot, preferred_element_type=jnp.float32)


def _elu(v):
    return jnp.where(v > 0, v, jnp.exp(jnp.minimum(v, 0.0)) - 1.0)


def _pre1_body(x_ref, W1_ref, a1s_ref, a1d_ref, as_ref, ad_ref, c_ref, m_ref):
    i = pl.program_id(0)
    us = _MM(W1_ref[...], a1s_ref[...])
    ud = _MM(W1_ref[...], a1d_ref[...])
    xb = x_ref[...]
    asb = _MM(xb, us)
    adb = _MM(xb, ud)
    as_ref[...] = asb
    ad_ref[...] = adb
    bs, bd = jnp.max(asb), jnp.max(adb)

    @pl.when(i == 0)
    def _():
        m_ref[0] = bs
        m_ref[1] = bd

    @pl.when(i > 0)
    def _():
        m_ref[0] = jnp.maximum(m_ref[0], bs)
        m_ref[1] = jnp.maximum(m_ref[1], bd)

    @pl.when(i == NB - 1)
    def _():
        s = m_ref[0] + m_ref[1]
        c = jnp.where(s >= 0, s, NEG * s)
        c_ref[...] = jnp.full((1, 128), c, jnp.float32)


def _pre1(x, W1, a1s, a1d):
    return pl.pallas_call(
        _pre1_body,
        grid=(NB,),
        in_specs=[
            pl.BlockSpec((BLK, D_IN), lambda i: (i, 0)),
            pl.BlockSpec((D_IN, H1), lambda i: (0, 0)),
            pl.BlockSpec((H1, 1), lambda i: (0, 0)),
            pl.BlockSpec((H1, 1), lambda i: (0, 0)),
        ],
        out_specs=[
            pl.BlockSpec((BLK, 1), lambda i: (i, 0)),
            pl.BlockSpec((BLK, 1), lambda i: (i, 0)),
            pl.BlockSpec((1, 128), lambda i: (0, 0)),
        ],
        out_shape=[
            jax.ShapeDtypeStruct((N, 1), jnp.float32),
            jax.ShapeDtypeStruct((N, 1), jnp.float32),
            jax.ShapeDtypeStruct((1, 128), jnp.float32),
        ],
        scratch_shapes=[pltpu.SMEM((2,), jnp.float32)],
    )(x, W1, a1s[:, None], a1d[:, None])


def _mid1_body(agg_ref, den_ref, W1_ref, b1_ref, W2_ref, a2s_ref, a2d_ref,
               g2_ref, as_ref, ad_ref, c_ref, m_ref):
    i = pl.program_id(0)
    agg = agg_ref[0] + agg_ref[1]
    den = jnp.sum(den_ref[0], axis=0).reshape(BLK, 1) + 1e-16
    h = _elu(_MM(agg / den, W1_ref[...]) + b1_ref[...])
    g2 = _MM(h, W2_ref[...])
    g2_ref[...] = g2
    asb = _MM(g2, a2s_ref[...])
    adb = _MM(g2, a2d_ref[...])
    as_ref[...] = asb
    ad_ref[...] = adb
    bs, bd = jnp.max(asb), jnp.max(adb)

    @pl.when(i == 0)
    def _():
        m_ref[0] = bs
        m_ref[1] = bd

    @pl.when(i > 0)
    def _():
        m_ref[0] = jnp.maximum(m_ref[0], bs)
        m_ref[1] = jnp.maximum(m_ref[1], bd)

    @pl.when(i == NB - 1)
    def _():
        s = m_ref[0] + m_ref[1]
        c = jnp.where(s >= 0, s, NEG * s)
        c_ref[...] = jnp.full((1, 128), c, jnp.float32)


def _mid1(agg, den, W1, b1, W2, a2s, a2d):
    return pl.pallas_call(
        _mid1_body,
        grid=(NB,),
        in_specs=[
            pl.BlockSpec((2, BLK, D_IN), lambda i: (0, i, 0)),
            pl.BlockSpec((1, NW, BLK), lambda i: (i, 0, 0)),
            pl.BlockSpec((D_IN, H1), lambda i: (0, 0)),
            pl.BlockSpec((1, H1), lambda i: (0, 0)),
            pl.BlockSpec((H1, H2), lambda i: (0, 0)),
            pl.BlockSpec((H2, 1), lambda i: (0, 0)),
            pl.BlockSpec((H2, 1), lambda i: (0, 0)),
        ],
        out_specs=[
            pl.BlockSpec((BLK, H2), lambda i: (i, 0)),
            pl.BlockSpec((BLK, 1), lambda i: (i, 0)),
            pl.BlockSpec((BLK, 1), lambda i: (i, 0)),
            pl.BlockSpec((1, 128), lambda i: (0, 0)),
        ],
        out_shape=[
            jax.ShapeDtypeStruct((N, H2), jnp.float32),
            jax.ShapeDtypeStruct((N, 1), jnp.float32),
            jax.ShapeDtypeStruct((N, 1), jnp.float32),
            jax.ShapeDtypeStruct((1, 128), jnp.float32),
        ],
        scratch_shapes=[pltpu.SMEM((2,), jnp.float32)],
    )(agg, jnp.swapaxes(den.reshape(NW, NB, BLK), 0, 1), W1, b1[None, :], W2,
      a2s[:, None], a2d[:, None])


def _mid2_body(agg_ref, den_ref, b2_ref, W3_ref, a3s_ref, a3d_ref,
               g3_ref, as_ref, ad_ref, c_ref, m_ref):
    i = pl.program_id(0)
    agg = agg_ref[0] + agg_ref[1]
    den = jnp.sum(den_ref[0], axis=0).reshape(BLK, 1) + 1e-16
    h = _elu(agg / den + b2_ref[...])
    g3 = _MM(h, W3_ref[...])
    g3_ref[...] = g3
    asb = _MM(g3, a3s_ref[...])
    adb = _MM(g3, a3d_ref[...])
    as_ref[...] = asb
    ad_ref[...] = adb
    bs, bd = jnp.max(asb), jnp.max(adb)

    @pl.when(i == 0)
    def _():
        m_ref[0] = bs
        m_ref[1] = bd

    @pl.when(i > 0)
    def _():
        m_ref[0] = jnp.maximum(m_ref[0], bs)
        m_ref[1] = jnp.maximum(m_ref[1], bd)

    @pl.when(i == NB - 1)
    def _():
        s = m_ref[0] + m_ref[1]
        c = jnp.where(s >= 0, s, NEG * s)
        c_ref[...] = jnp.full((1, 128), c, jnp.float32)


def _mid2(agg, den, b2, W3p, a3sp, a3dp):
    return pl.pallas_call(
        _mid2_body,
        grid=(NB,),
        in_specs=[
            pl.BlockSpec((2, BLK, H2), lambda i: (0, i, 0)),
            pl.BlockSpec((1, NW, BLK), lambda i: (i, 0, 0)),
            pl.BlockSpec((1, H2), lambda i: (0, 0)),
            pl.BlockSpec((H2, 16), lambda i: (0, 0)),
            pl.BlockSpec((16, 1), lambda i: (0, 0)),
            pl.BlockSpec((16, 1), lambda i: (0, 0)),
        ],
        out_specs=[
            pl.BlockSpec((BLK, 16), lambda i: (i, 0)),
            pl.BlockSpec((BLK, 1), lambda i: (i, 0)),
            pl.BlockSpec((BLK, 1), lambda i: (i, 0)),
            pl.BlockSpec((1, 128), lambda i: (0, 0)),
        ],
        out_shape=[
            jax.ShapeDtypeStruct((N, 16), jnp.float32),
            jax.ShapeDtypeStruct((N, 1), jnp.float32),
            jax.ShapeDtypeStruct((N, 1), jnp.float32),
            jax.ShapeDtypeStruct((1, 128), jnp.float32),
        ],
        scratch_shapes=[pltpu.SMEM((2,), jnp.float32)],
    )(agg, jnp.swapaxes(den.reshape(NW, NB, BLK), 0, 1), b2[None, :], W3p,
      a3sp[:, None], a3dp[:, None])


def _fin_body(agg_ref, den_ref, b3_ref, out_ref):
    agg = agg_ref[0] + agg_ref[1]
    den = jnp.sum(den_ref[0], axis=0).reshape(BLK, 1) + 1e-16
    out_ref[...] = agg / den + b3_ref[...]


def _fin(agg, den, b3p):
    return pl.pallas_call(
        _fin_body,
        grid=(NB,),
        in_specs=[
            pl.BlockSpec((2, BLK, 16), lambda i: (0, i, 0)),
            pl.BlockSpec((1, NW, BLK), lambda i: (i, 0, 0)),
            pl.BlockSpec((1, 16), lambda i: (0, 0)),
        ],
        out_specs=pl.BlockSpec((BLK, 16), lambda i: (i, 0)),
        out_shape=jax.ShapeDtypeStruct((N, 16), jnp.float32),
    )(agg, jnp.swapaxes(den.reshape(NW, NB, BLK), 0, 1), b3p[None, :])


def _sc_edge_layer(F):
    mesh = plsc.VectorSubcoreMesh(core_axis_name="c", subcore_axis_name="s")
    rpt = N // NSUB

    @functools.partial(
        pl.kernel,
        out_type=[
            jax.ShapeDtypeStruct((NCORE, N, F), jnp.float32),
            jax.ShapeDtypeStruct((NW, N), jnp.float32),
        ],
        mesh=mesh,
        compiler_params=pltpu.CompilerParams(use_tc_tiling_on_sc=False,
                                             needs_layout_passes=False),
        scratch_types=[
            pltpu.VMEM_SHARED((N, F), jnp.float32),
            pltpu.VMEM((N + 16,), jnp.float32),
            pltpu.VMEM((N + 16,), jnp.float32),
            pltpu.VMEM((N,), jnp.float32),
            pltpu.VMEM((16,), jnp.float32),
            pltpu.VMEM((CH,), jnp.int32),
            pltpu.VMEM((CH,), jnp.int32),
            pltpu.VMEM((CH,), jnp.int32),
            pltpu.VMEM((CH,), jnp.int32),
            pltpu.VMEM((CH,), jnp.float32),
            pltpu.VMEM((CH, F), jnp.float32),
            pltpu.VMEM((CH, F), jnp.float32),
            pltpu.SemaphoreType.DMA,
            pltpu.SemaphoreType.DMA,
            pltpu.SemaphoreType.DMA,
            pltpu.SemaphoreType.DMA,
        ],
    )
    def sc_fn(src_hbm, dst_hbm, as_hbm, ad_hbm, c_hbm, g_hbm, zrows_hbm,
              zden_hbm, agg_out, den_out,
              agg_sh, as_v, ad_v, den_v, c_v, src_a, src_b, dst_a, dst_b,
              w_v, rows_a, rows_b,
              sem_ra, sem_rb, sem_sa, sem_sb):
        cid = lax.axis_index("c")
        sid = lax.axis_index("s")
        wid = sid * NCORE + cid
        base = wid * EPT

        pltpu.sync_copy(as_hbm, as_v)
        pltpu.sync_copy(ad_hbm, ad_v)
        pltpu.sync_copy(c_hbm, c_v)
        pltpu.sync_copy(zden_hbm, den_v)
        pltpu.sync_copy(zrows_hbm.at[pl.ds(sid * rpt, rpt)],
                        agg_sh.at[pl.ds(sid * rpt, rpt)])
        plsc.subcore_barrier()

        cvec = c_v[...]

        def prefetch(k, sv, dv, rv, sem_r):
            off = base + k * CH
            pltpu.sync_copy(src_hbm.at[pl.ds(off, CH)], sv)
            pltpu.sync_copy(dst_hbm.at[pl.ds(off, CH)], dv)
            pltpu.async_copy(g_hbm.at[sv], rv, sem_r)

        def weight_pass(sv, dv):
            for g16 in range(CH // LANES):
                sl = pl.ds(g16 * LANES, LANES)
                didx = dv[sl]
                s = (plsc.load_gather(as_v, [sv[sl]])
                     + plsc.load_gather(ad_v, [didx]))
                e = jnp.where(s >= 0, s, NEG * s)
                w = jnp.exp(e - cvec)
                w_v[sl] = w
                plsc.addupdate_scatter(den_v, [didx], w)

        def scale_scatter(sv, dv, rv, sem_r, sem_s):
            pltpu.make_async_copy(g_hbm.at[sv], rv, sem_r).wait()
            for row in range(CH):
                ws = plsc.load_gather(
                    w_v, [jnp.full((LANES,), row, jnp.int32)])
                for f in range(F // LANES):
                    fsl = pl.ds(f * LANES, LANES)
                    rv[row, fsl] = rv[row, fsl] * ws
            pltpu.async_copy(rv, agg_sh.at[dv], sem_s, add=True)

        def wait_scatter(dv, rv, sem_s):
            pltpu.make_async_copy(rv, agg_sh.at[dv], sem_s).wait()

        prefetch(0, src_a, dst_a, rows_a, sem_ra)

        def pair(j, carry):
            k0 = 2 * j

            @pl.when(j > 0)
            def _():
                wait_scatter(dst_b, rows_b, sem_sb)

            prefetch(k0 + 1, src_b, dst_b, rows_b, sem_rb)
            weight_pass(src_a, dst_a)
            scale_scatter(src_a, dst_a, rows_a, sem_ra, sem_sa)

            weight_pass(src_b, dst_b)

            @pl.when(k0 + 2 < NCHUNK)
            def _():
                wait_scatter(dst_a, rows_a, sem_sa)
                prefetch(k0 + 2, src_a, dst_a, rows_a, sem_ra)

            scale_scatter(src_b, dst_b, rows_b, sem_rb, sem_sb)
            return carry

        lax.fori_loop(0, NCHUNK // 2, pair, 0)
        wait_scatter(dst_a, rows_a, sem_sa)
        wait_scatter(dst_b, rows_b, sem_sb)

        pltpu.sync_copy(den_v, den_out.at[wid])
        plsc.subcore_barrier()

        rsl = pl.ds(sid * rpt, rpt)
        pltpu.sync_copy(agg_sh.at[rsl], agg_out.at[cid, rsl])

    return sc_fn


_sc128 = _sc_edge_layer(128)
_sc16 = _sc_edge_layer(16)


_SENT = -1e30


def _pad_tables(asv, adv):
    a = jnp.concatenate([asv, jnp.full((16,), _SENT, jnp.float32)])
    b = jnp.concatenate([adv, jnp.zeros((16,), jnp.float32)])
    return a, b


def _pad_g(g, f):
    return jnp.concatenate([g, jnp.zeros((16, f), jnp.float32)])


def kernel(x, edge_index, W1, a1s, a1d, b1, W2, a2s, a2d, b2,
           W3, a3s, a3d, b3):
    loop = jnp.arange(N, dtype=edge_index.dtype)
    pad_src = jnp.full((EPAD - ETOT,), N, edge_index.dtype)
    pad_dst = jnp.zeros((EPAD - ETOT,), edge_index.dtype)
    src = jnp.concatenate([edge_index[0], loop, pad_src])
    dst = jnp.concatenate([edge_index[1], loop, pad_dst])
    packed = jnp.sort((dst << 14) | src)
    totc = EPAD // CH
    packed = packed.reshape(CH, totc).T.reshape(EPAD)
    src = packed & 16383
    dst = packed >> 14
    z128 = jnp.zeros((N, 128), jnp.float32)
    z16 = jnp.zeros((N, 16), jnp.float32)
    zden = jnp.zeros((N,), jnp.float32)

    as1, ad1, c1 = _pre1(x, W1, a1s, a1d)
    asp, adp = _pad_tables(as1.reshape(N), ad1.reshape(N))
    agg1, den1 = _sc128(src, dst, asp, adp,
                        c1.reshape(128)[:16], _pad_g(x, 128), z128, zden)

    g2, as2, ad2, c2 = _mid1(agg1, den1, W1, b1, W2, a2s, a2d)
    asp, adp = _pad_tables(as2.reshape(N), ad2.reshape(N))
    agg2, den2 = _sc128(src, dst, asp, adp,
                        c2.reshape(128)[:16], _pad_g(g2, 128), z128, zden)

    W3p = jnp.pad(W3, ((0, 0), (0, 16 - NCLS)))
    a3sp = jnp.pad(a3s, (0, 16 - NCLS))
    a3dp = jnp.pad(a3d, (0, 16 - NCLS))
    b3p = jnp.pad(b3, (0, 16 - NCLS))
    g3, as3, ad3, c3 = _mid2(agg2, den2, b2, W3p, a3sp, a3dp)
    asp, adp = _pad_tables(as3.reshape(N), ad3.reshape(N))
    agg3, den3 = _sc16(src, dst, asp, adp,
                       c3.reshape(128)[:16], _pad_g(g3, 16), z16, zden)

    out = _fin(agg3, den3, b3p)
    return out[:, :NCLS]
